# Initial kernel scaffold; baseline (speedup 1.0000x reference)
import jax
import jax.numpy as jnp
from jax.experimental import pallas as pl


def kernel(cos_theta, cos_theta_m, label):
    raise NotImplementedError("write your pallas kernel here")


# speedup vs baseline: 1.1106x; 1.1106x over previous
import functools

import jax
import jax.numpy as jnp
from jax import lax
from jax.experimental import pallas as pl
from jax.experimental.pallas import tpu as pltpu
from jax.experimental.pallas import tpu_sc as plsc

S = 64.0
B = 1024
C = 100000


_NC = 2
_NS = 16
_L = 16
_NW = _NC * _NS
_B_PER_W = B // _NW


def _sc_gather_kernel(ctm_hbm, label_hbm, out_hbm, idx_v, val_v, sem):
    wid = lax.axis_index("s") * _NC + lax.axis_index("c")
    base = wid * _B_PER_W
    pltpu.sync_copy(label_hbm.at[pl.ds(base, _B_PER_W)], idx_v)
    for j in range(_B_PER_W // _L):
        lbl = idx_v[pl.ds(j * _L, _L)]
        rows = lax.iota(jnp.int32, _L) + (base + j * _L)
        idx_v[pl.ds(j * _L, _L)] = rows * C + lbl
    pltpu.async_copy(ctm_hbm.at[idx_v], val_v, sem).wait()
    pltpu.sync_copy(val_v, out_hbm.at[pl.ds(base, _B_PER_W)])


def _sc_gather(ctm_flat, label):
    mesh = plsc.VectorSubcoreMesh(core_axis_name="c", subcore_axis_name="s")
    fn = functools.partial(
        pl.kernel,
        mesh=mesh,
        out_type=jax.ShapeDtypeStruct((B,), jnp.float32),
        scratch_types=[
            pltpu.VMEM((_B_PER_W,), jnp.int32),
            pltpu.VMEM((_B_PER_W,), jnp.float32),
            pltpu.SemaphoreType.DMA,
        ],
    )(_sc_gather_kernel)
    return fn(ctm_flat, label)


_BLK_W = 512
_CB = -(-C // _BLK_W)


def _tc_body(cos_ref, lab_ref, m_ref, out_ref, acc_ref):
    cb = pl.program_id(0)

    @pl.when(cb == 0)
    def _init():
        acc_ref[...] = jnp.zeros_like(acc_ref)
        out_ref[...] = jnp.zeros_like(out_ref)

    col = lax.broadcasted_iota(jnp.int32, (B, _BLK_W), 1) + cb * _BLK_W
    drop = (col == lab_ref[...]) | (col >= C)
    x = jnp.where(drop, -jnp.inf, cos_ref[...] * S)
    acc_ref[...] += jnp.sum(jnp.exp(x), axis=1, keepdims=True)

    @pl.when(cb == pl.num_programs(0) - 1)
    def _fini():
        sm = m_ref[...] * S
        total = acc_ref[...] + jnp.exp(sm)
        li = jnp.log(total) - sm
        out_ref[...] = jnp.sum(li, axis=0, keepdims=True) / B


def _tc_loss(cos_theta, label2d, m2d, interpret=False):
    return pl.pallas_call(
        _tc_body,
        grid=(_CB,),
        in_specs=[
            pl.BlockSpec((B, _BLK_W), lambda cb: (0, cb)),
            pl.BlockSpec((B, 1), lambda cb: (0, 0)),
            pl.BlockSpec((B, 1), lambda cb: (0, 0)),
        ],
        out_specs=pl.BlockSpec((1, 1), lambda cb: (0, 0)),
        out_shape=jax.ShapeDtypeStruct((1, 1), jnp.float32),
        scratch_shapes=[pltpu.VMEM((B, 1), jnp.float32)],
        compiler_params=pltpu.CompilerParams(
            dimension_semantics=("arbitrary",),
        ),
        interpret=interpret,
    )(cos_theta, label2d, m2d)


def kernel(cos_theta, cos_theta_m, label):
    label = label.astype(jnp.int32)
    m = _sc_gather(cos_theta_m.reshape(B * C), label)
    out = _tc_loss(cos_theta, label.reshape(B, 1), m.reshape(B, 1))
    return out[0, 0]
